# baseline (speedup 1.0000x reference)
import functools

import jax
import jax.numpy as jnp
from jax import lax
from jax.experimental import pallas as pl
from jax.experimental.pallas import tpu as pltpu
from jax.experimental.pallas import tpu_sc as plsc

N = 10000
NP = 10240
E = 320000
C = 128
H = 64
K = 32
HD = 128

NW = 32
CW = 128
NCH = 80
EP = NW * NCH * CW
RPS = NP // 16
RB = 1280

_mesh = plsc.VectorSubcoreMesh(core_axis_name="c", subcore_axis_name="s")


@functools.partial(
    pl.kernel,
    mesh=_mesh,
    out_type=jax.ShapeDtypeStruct((2, NP, HD), jnp.float32),
    scratch_types=[
        pltpu.VMEM((8, CW), jnp.int32),
        pltpu.VMEM((8, CW), jnp.int32),
        pltpu.VMEM((8, CW), jnp.int32),
        pltpu.VMEM((8, CW), jnp.int32),
        pltpu.VMEM((CW,), jnp.float32),
        pltpu.VMEM((CW,), jnp.float32),
        pltpu.VMEM((CW,), jnp.float32),
        pltpu.VMEM((CW,), jnp.float32),
        pltpu.VMEM((CW, HD), jnp.float32),
        pltpu.VMEM((CW, HD), jnp.float32),
        pltpu.VMEM_SHARED((NP, HD), jnp.float32),
        pltpu.SemaphoreType.DMA,
        pltpu.SemaphoreType.DMA,
        pltpu.SemaphoreType.DMA,
        pltpu.SemaphoreType.DMA,
        pltpu.SemaphoreType.DMA,
        pltpu.SemaphoreType.DMA,
        pltpu.SemaphoreType.DMA,
    ],
)
def _sc_msgpass(h_hbm, pk_hbm, wp_hbm, zeros_hbm, out_hbm,
                pk0, pk1, pk2, pk3, w0, w_1, w2, w3, rin0, rin1, acc,
                is0, is1, is2, is3, gs0, gs1, ssem):
    c = lax.axis_index("c")
    s = lax.axis_index("s")
    w = c * 16 + s
    e0 = jnp.where(lax.iota(jnp.int32, 16) == 0, 1.0, 0.0).astype(jnp.float32)
    pkb = [pk0, pk1, pk2, pk3]
    wb = [w0, w_1, w2, w3]
    rinb = [rin0, rin1]
    isem = [is0, is1, is2, is3]
    gsem = [gs0, gs1]

    for k in range(RPS // CW):
        pltpu.sync_copy(zeros_hbm.at[pl.ds(s * RPS + k * CW, CW)],
                        acc.at[pl.ds(s * RPS + k * CW, CW)])
    plsc.subcore_barrier()

    base = w * NCH

    def fire_idx(ch, slot):
        pltpu.async_copy(pk_hbm.at[base + ch], pkb[slot], isem[slot])
        pltpu.async_copy(wp_hbm.at[pl.ds((base + ch) * CW, CW)],
                         wb[slot], isem[slot])

    def wait_idx(slot):
        pltpu.make_async_copy(pk_hbm.at[0], pkb[slot], isem[slot]).wait()
        pltpu.make_async_copy(wp_hbm.at[pl.ds(0, CW)], wb[slot],
                              isem[slot]).wait()

    def scale(rs, islot):
        def rowfn(r16, carry2):
            wg = wb[islot][pl.ds(r16 * 16, 16)]
            b2 = r16 * 16
            for ll in range(16):
                row = b2 + ll
                ws = wg[ll]
                for cc in range(H // 16):
                    rinb[rs][row, pl.ds(cc * 16, 16)] = (
                        rinb[rs][row, pl.ds(cc * 16, 16)] * ws)
                rinb[rs][row, pl.ds(H, 16)] = e0 * ws
            return carry2
        lax.fori_loop(0, CW // 16, rowfn, 0)

    fire_idx(0, 0)
    fire_idx(1, 1)
    wait_idx(0)
    pltpu.async_copy(h_hbm.at[pkb[0].at[0]], rinb[0], gsem[0])

    def group(g, carry):
        for b in range(4):
            ch = g * 4 + b
            p = b % 2
            q = 1 - p
            i0 = b
            i1 = (b + 1) % 4
            i2 = (b + 2) % 4
            i3 = (b + 3) % 4

            @pl.when(ch + 2 < NCH)
            def _fire_idx2():
                fire_idx(ch + 2, i2)

            @pl.when(ch >= 1)
            def _wait_prev_scatter():
                pltpu.make_async_copy(rinb[q], acc.at[pkb[i3].at[1]],
                                      ssem).wait()

            @pl.when(ch + 1 < NCH)
            def _prefetch_gather():
                wait_idx(i1)
                pltpu.async_copy(h_hbm.at[pkb[i1].at[0]], rinb[q], gsem[q])

            pltpu.make_async_copy(h_hbm.at[pkb[i0].at[0]], rinb[p],
                                  gsem[p]).wait()
            scale(p, i0)
            pltpu.async_copy(rinb[p], acc.at[pkb[i0].at[1]], ssem, add=True)
        return carry

    lax.fori_loop(0, NCH // 4, group, 0)
    pltpu.make_async_copy(rinb[1], acc.at[pkb[3].at[1]], ssem).wait()
    plsc.subcore_barrier()
    for k in range(RPS // CW):
        pltpu.sync_copy(acc.at[pl.ds(s * RPS + k * CW, CW)],
                        out_hbm.at[c, pl.ds(s * RPS + k * CW, CW)])


def _enc_body(x_ref, w0_ref, b0_ref, w1_ref, b1_ref, o_ref):
    h = jax.nn.relu(jnp.dot(x_ref[...], w0_ref[...],
                            preferred_element_type=jnp.float32) + b0_ref[...])
    h2 = jax.nn.relu(jnp.dot(h, w1_ref[...],
                             preferred_element_type=jnp.float32) + b1_ref[...])
    o_ref[:, :H] = h2
    o_ref[:, H:] = jnp.zeros_like(o_ref[:, H:])


def _encoder(x, w0, b0, w1, b1):
    grid = (NP // RB,)
    return pl.pallas_call(
        _enc_body,
        grid=grid,
        in_specs=[
            pl.BlockSpec((RB, C), lambda i: (i, 0)),
            pl.BlockSpec((C, H), lambda i: (0, 0)),
            pl.BlockSpec((1, H), lambda i: (0, 0)),
            pl.BlockSpec((H, H), lambda i: (0, 0)),
            pl.BlockSpec((1, H), lambda i: (0, 0)),
        ],
        out_specs=pl.BlockSpec((RB, HD), lambda i: (i, 0)),
        out_shape=jax.ShapeDtypeStruct((NP, HD), jnp.float32),
    )(x, w0, b0, w1, b1)


def _update_body(include_u, msum_ref, h_ref, cst_ref, w0_ref, b0_ref,
                 w1_ref, b1_ref, g_ref, bb_ref, tw_ref, tb_ref,
                 h_out_ref, *maybe_u_ref):
    cst = cst_ref[0, 0]
    p = msum_ref[0] + msum_ref[1]
    mm = p[:, :H] * cst
    dd = p[:, H:H + 1] * cst
    neigh = mm / (dd + 1e-8)
    hu = jnp.dot(jax.nn.relu(jnp.dot(neigh, w0_ref[...],
                                     preferred_element_type=jnp.float32)
                             + b0_ref[...]),
                 w1_ref[...], preferred_element_type=jnp.float32) + b1_ref[...]
    t = h_ref[:, :H] + hu
    m = jnp.mean(t, axis=-1, keepdims=True)
    v = jnp.mean((t - m) * (t - m), axis=-1, keepdims=True)
    hn = (t - m) / jnp.sqrt(v + 1e-5) * g_ref[...] + bb_ref[...]
    h_out_ref[:, :H] = hn
    h_out_ref[:, H:] = jnp.zeros_like(h_out_ref[:, H:])
    if include_u:
        z = jnp.dot(hn, tw_ref[...],
                    preferred_element_type=jnp.float32) + tb_ref[...]
        maybe_u_ref[0][...] = (jnp.log1p(jnp.exp(-jnp.abs(z)))
                               + jnp.maximum(z, 0.0))


def _update(msum, h, cst, w0, b0, w1, b1, g, bb, tw, tb, include_u):
    grid = (NP // RB,)
    out_shape = [jax.ShapeDtypeStruct((NP, HD), jnp.float32)]
    out_specs = [pl.BlockSpec((RB, HD), lambda i: (i, 0))]
    if include_u:
        out_shape.append(jax.ShapeDtypeStruct((NP, K), jnp.float32))
        out_specs.append(pl.BlockSpec((RB, K), lambda i: (i, 0)))
    return pl.pallas_call(
        functools.partial(_update_body, include_u),
        grid=grid,
        in_specs=[
            pl.BlockSpec((2, RB, HD), lambda i: (0, i, 0)),
            pl.BlockSpec((RB, HD), lambda i: (i, 0)),
            pl.BlockSpec((1, 1), lambda i: (0, 0), memory_space=pltpu.SMEM),
            pl.BlockSpec((H, H), lambda i: (0, 0)),
            pl.BlockSpec((1, H), lambda i: (0, 0)),
            pl.BlockSpec((H, H), lambda i: (0, 0)),
            pl.BlockSpec((1, H), lambda i: (0, 0)),
            pl.BlockSpec((1, H), lambda i: (0, 0)),
            pl.BlockSpec((1, H), lambda i: (0, 0)),
            pl.BlockSpec((H, K), lambda i: (0, 0)),
            pl.BlockSpec((1, K), lambda i: (0, 0)),
        ],
        out_specs=out_specs,
        out_shape=out_shape,
    )(msum, h, cst, w0, b0, w1, b1, g, bb, tw, tb)


def _wout_body(bw_ref, cst_ref, o_ref):
    o_ref[...] = bw_ref[...] * cst_ref[0, 0]


def _wout(base_w, cst):
    bw = base_w.reshape(E // 128, 128)
    out = pl.pallas_call(
        _wout_body,
        in_specs=[
            pl.BlockSpec((E // 128, 128), lambda: (0, 0)),
            pl.BlockSpec((1, 1), lambda: (0, 0), memory_space=pltpu.SMEM),
        ],
        out_specs=pl.BlockSpec((E // 128, 128), lambda: (0, 0)),
        out_shape=jax.ShapeDtypeStruct((E // 128, 128), jnp.float32),
    )(bw, cst)
    return out.reshape(E)


def kernel(x_common, src, dst, base_w, enc_W0, enc_b0, enc_W1, enc_b1,
           u0_W0, u0_b0, u0_W1, u0_b1, u1_W0, u1_b0, u1_W1, u1_b1,
           ln0_g, ln0_b, ln1_g, ln1_b, g_W0, g_b0, g_W1, g_b1,
           rho_raw, toU_W, toU_b):
    f32 = jnp.float32
    gate = jax.nn.sigmoid(g_b1[0])
    rho0 = jax.nn.sigmoid(rho_raw[0])
    cst = (gate * rho0 * rho0).astype(f32).reshape(1, 1)

    pad = EP - E
    pad_idx = (jnp.arange(pad, dtype=jnp.int32) * 37) % N
    srcp = jnp.concatenate([src.astype(jnp.int32), pad_idx])
    dstp = jnp.concatenate([dst.astype(jnp.int32), pad_idx])
    wp = jnp.concatenate([base_w.astype(f32), jnp.zeros((pad,), f32)])
    nrows = NW * NCH
    pk = jnp.concatenate(
        [srcp.reshape(nrows, 1, CW),
         dstp.reshape(nrows, 1, CW),
         jnp.zeros((nrows, 6, CW), jnp.int32)], axis=1)
    zeros_acc = jnp.zeros((NP, HD), f32)

    xp = jnp.zeros((NP, C), f32).at[:N].set(x_common.astype(f32))
    eb0 = enc_b0.reshape(1, H)
    eb1 = enc_b1.reshape(1, H)

    h0 = _encoder(xp, enc_W0, eb0, enc_W1, eb1)

    msum1 = _sc_msgpass(h0, pk, wp, zeros_acc)
    h1 = _update(msum1, h0, cst, u0_W0, u0_b0.reshape(1, H),
                 u0_W1, u0_b1.reshape(1, H), ln0_g.reshape(1, H),
                 ln0_b.reshape(1, H), toU_W, toU_b.reshape(1, K),
                 include_u=False)[0]

    msum2 = _sc_msgpass(h1, pk, wp, zeros_acc)
    h2, U = _update(msum2, h1, cst, u1_W0, u1_b0.reshape(1, H),
                    u1_W1, u1_b1.reshape(1, H), ln1_g.reshape(1, H),
                    ln1_b.reshape(1, H), toU_W, toU_b.reshape(1, K),
                    include_u=True)

    w_out = _wout(base_w.astype(f32), cst)
    return U[:N], h2[:N, :H], w_out


# speedup vs baseline: 1.0160x; 1.0160x over previous
import functools

import jax
import jax.numpy as jnp
from jax import lax
from jax.experimental import pallas as pl
from jax.experimental.pallas import tpu as pltpu
from jax.experimental.pallas import tpu_sc as plsc

N = 10000
NP = 10240
E = 320000
C = 128
H = 64
K = 32
HD = 128

NW = 32
CW = 128
NCH = 80
EP = NW * NCH * CW
RPS = NP // 16
RB = 1280

_mesh = plsc.VectorSubcoreMesh(core_axis_name="c", subcore_axis_name="s")


@functools.partial(
    pl.kernel,
    mesh=_mesh,
    out_type=jax.ShapeDtypeStruct((2, NP, HD), jnp.float32),
    scratch_types=[
        pltpu.VMEM((CW,), jnp.int32),
        pltpu.VMEM((CW,), jnp.int32),
        pltpu.VMEM((CW,), jnp.int32),
        pltpu.VMEM((CW,), jnp.int32),
        pltpu.VMEM((CW,), jnp.int32),
        pltpu.VMEM((CW,), jnp.int32),
        pltpu.VMEM((CW,), jnp.int32),
        pltpu.VMEM((CW,), jnp.int32),
        pltpu.VMEM((CW,), jnp.float32),
        pltpu.VMEM((CW,), jnp.float32),
        pltpu.VMEM((CW,), jnp.float32),
        pltpu.VMEM((CW,), jnp.float32),
        pltpu.VMEM((CW, HD), jnp.float32),
        pltpu.VMEM((CW, HD), jnp.float32),
        pltpu.VMEM_SHARED((NP, HD), jnp.float32),
        pltpu.SemaphoreType.DMA,
        pltpu.SemaphoreType.DMA,
        pltpu.SemaphoreType.DMA,
        pltpu.SemaphoreType.DMA,
        pltpu.SemaphoreType.DMA,
        pltpu.SemaphoreType.DMA,
        pltpu.SemaphoreType.DMA,
    ],
)
def _sc_msgpass(h_hbm, srcp, dstp, wp, zeros_hbm, out_hbm,
                src0, src_1, src2, src3, dst0, dst_1, dst2, dst3,
                w0, w_1, w2, w3, rin0, rin1, acc,
                is0, is1, is2, is3, gs0, gs1, ssem):
    c = lax.axis_index("c")
    s = lax.axis_index("s")
    w = c * 16 + s
    e0 = jnp.where(lax.iota(jnp.int32, 16) == 0, 1.0, 0.0).astype(jnp.float32)
    srcb = [src0, src_1, src2, src3]
    dstb = [dst0, dst_1, dst2, dst3]
    wb = [w0, w_1, w2, w3]
    rinb = [rin0, rin1]
    isem = [is0, is1, is2, is3]
    gsem = [gs0, gs1]

    for k in range(RPS // CW):
        pltpu.sync_copy(zeros_hbm.at[pl.ds(s * RPS + k * CW, CW)],
                        acc.at[pl.ds(s * RPS + k * CW, CW)])
    plsc.subcore_barrier()

    base = w * (NCH * CW)

    def fire_idx(ch, slot):
        off = base + ch * CW
        pltpu.async_copy(srcp.at[pl.ds(off, CW)], srcb[slot], isem[slot])
        pltpu.async_copy(dstp.at[pl.ds(off, CW)], dstb[slot], isem[slot])
        pltpu.async_copy(wp.at[pl.ds(off, CW)], wb[slot], isem[slot])

    def wait_idx(slot):
        pltpu.make_async_copy(srcp.at[pl.ds(0, CW)], srcb[slot],
                              isem[slot]).wait()
        pltpu.make_async_copy(dstp.at[pl.ds(0, CW)], dstb[slot],
                              isem[slot]).wait()
        pltpu.make_async_copy(wp.at[pl.ds(0, CW)], wb[slot],
                              isem[slot]).wait()

    def scale(rs, ws_slot):
        def rowfn(r16, carry2):
            wg = wb[ws_slot][pl.ds(r16 * 16, 16)]
            b2 = r16 * 16
            for ll in range(16):
                row = b2 + ll
                ws = wg[ll]
                for cc in range(H // 16):
                    rinb[rs][row, pl.ds(cc * 16, 16)] = (
                        rinb[rs][row, pl.ds(cc * 16, 16)] * ws)
                rinb[rs][row, pl.ds(H, 16)] = e0 * ws
            return carry2
        lax.fori_loop(0, CW // 16, rowfn, 0)

    fire_idx(0, 0)
    fire_idx(1, 1)
    wait_idx(0)
    pltpu.async_copy(h_hbm.at[srcb[0]], rinb[0], gsem[0])

    def group(g, carry):
        for b in range(4):
            ch = g * 4 + b
            p = b % 2
            q = 1 - p
            i0 = b
            i1 = (b + 1) % 4
            i2 = (b + 2) % 4

            @pl.when(ch + 2 < NCH)
            def _fire_idx2():
                fire_idx(ch + 2, i2)

            i3 = (b + 3) % 4

            @pl.when(ch >= 1)
            def _wait_prev_scatter():
                pltpu.make_async_copy(rinb[q], acc.at[dstb[i3]],
                                      ssem).wait()

            @pl.when(ch + 1 < NCH)
            def _prefetch_gather():
                wait_idx(i1)
                pltpu.async_copy(h_hbm.at[srcb[i1]], rinb[q], gsem[q])

            pltpu.make_async_copy(h_hbm.at[srcb[i0]], rinb[p],
                                  gsem[p]).wait()
            scale(p, i0)
            pltpu.async_copy(rinb[p], acc.at[dstb[i0]], ssem, add=True)
        return carry

    lax.fori_loop(0, NCH // 4, group, 0)
    pltpu.make_async_copy(rinb[1], acc.at[dstb[3]], ssem).wait()
    plsc.subcore_barrier()
    for k in range(RPS // CW):
        pltpu.sync_copy(acc.at[pl.ds(s * RPS + k * CW, CW)],
                        out_hbm.at[c, pl.ds(s * RPS + k * CW, CW)])


def _enc_body(x_ref, w0_ref, b0_ref, w1_ref, b1_ref, o_ref):
    h = jax.nn.relu(jnp.dot(x_ref[...], w0_ref[...],
                            preferred_element_type=jnp.float32) + b0_ref[...])
    h2 = jax.nn.relu(jnp.dot(h, w1_ref[...],
                             preferred_element_type=jnp.float32) + b1_ref[...])
    o_ref[:, :H] = h2
    o_ref[:, H:] = jnp.zeros_like(o_ref[:, H:])


def _encoder(x, w0, b0, w1, b1):
    grid = (NP // RB,)
    return pl.pallas_call(
        _enc_body,
        grid=grid,
        in_specs=[
            pl.BlockSpec((RB, C), lambda i: (i, 0)),
            pl.BlockSpec((C, H), lambda i: (0, 0)),
            pl.BlockSpec((1, H), lambda i: (0, 0)),
            pl.BlockSpec((H, H), lambda i: (0, 0)),
            pl.BlockSpec((1, H), lambda i: (0, 0)),
        ],
        out_specs=pl.BlockSpec((RB, HD), lambda i: (i, 0)),
        out_shape=jax.ShapeDtypeStruct((NP, HD), jnp.float32),
    )(x, w0, b0, w1, b1)


def _update_body(include_u, msum_ref, h_ref, cst_ref, w0_ref, b0_ref,
                 w1_ref, b1_ref, g_ref, bb_ref, tw_ref, tb_ref,
                 h_out_ref, *maybe_u_ref):
    cst = cst_ref[0, 0]
    p = msum_ref[0] + msum_ref[1]
    mm = p[:, :H] * cst
    dd = p[:, H:H + 1] * cst
    neigh = mm / (dd + 1e-8)
    hu = jnp.dot(jax.nn.relu(jnp.dot(neigh, w0_ref[...],
                                     preferred_element_type=jnp.float32)
                             + b0_ref[...]),
                 w1_ref[...], preferred_element_type=jnp.float32) + b1_ref[...]
    t = h_ref[:, :H] + hu
    m = jnp.mean(t, axis=-1, keepdims=True)
    v = jnp.mean((t - m) * (t - m), axis=-1, keepdims=True)
    hn = (t - m) / jnp.sqrt(v + 1e-5) * g_ref[...] + bb_ref[...]
    h_out_ref[:, :H] = hn
    h_out_ref[:, H:] = jnp.zeros_like(h_out_ref[:, H:])
    if include_u:
        z = jnp.dot(hn, tw_ref[...],
                    preferred_element_type=jnp.float32) + tb_ref[...]
        maybe_u_ref[0][...] = (jnp.log1p(jnp.exp(-jnp.abs(z)))
                               + jnp.maximum(z, 0.0))


def _update(msum, h, cst, w0, b0, w1, b1, g, bb, tw, tb, include_u):
    grid = (NP // RB,)
    out_shape = [jax.ShapeDtypeStruct((NP, HD), jnp.float32)]
    out_specs = [pl.BlockSpec((RB, HD), lambda i: (i, 0))]
    if include_u:
        out_shape.append(jax.ShapeDtypeStruct((NP, K), jnp.float32))
        out_specs.append(pl.BlockSpec((RB, K), lambda i: (i, 0)))
    return pl.pallas_call(
        functools.partial(_update_body, include_u),
        grid=grid,
        in_specs=[
            pl.BlockSpec((2, RB, HD), lambda i: (0, i, 0)),
            pl.BlockSpec((RB, HD), lambda i: (i, 0)),
            pl.BlockSpec((1, 1), lambda i: (0, 0), memory_space=pltpu.SMEM),
            pl.BlockSpec((H, H), lambda i: (0, 0)),
            pl.BlockSpec((1, H), lambda i: (0, 0)),
            pl.BlockSpec((H, H), lambda i: (0, 0)),
            pl.BlockSpec((1, H), lambda i: (0, 0)),
            pl.BlockSpec((1, H), lambda i: (0, 0)),
            pl.BlockSpec((1, H), lambda i: (0, 0)),
            pl.BlockSpec((H, K), lambda i: (0, 0)),
            pl.BlockSpec((1, K), lambda i: (0, 0)),
        ],
        out_specs=out_specs,
        out_shape=out_shape,
    )(msum, h, cst, w0, b0, w1, b1, g, bb, tw, tb)


def _wout_body(bw_ref, cst_ref, o_ref):
    o_ref[...] = bw_ref[...] * cst_ref[0, 0]


def _wout(base_w, cst):
    bw = base_w.reshape(E // 128, 128)
    out = pl.pallas_call(
        _wout_body,
        in_specs=[
            pl.BlockSpec((E // 128, 128), lambda: (0, 0)),
            pl.BlockSpec((1, 1), lambda: (0, 0), memory_space=pltpu.SMEM),
        ],
        out_specs=pl.BlockSpec((E // 128, 128), lambda: (0, 0)),
        out_shape=jax.ShapeDtypeStruct((E // 128, 128), jnp.float32),
    )(bw, cst)
    return out.reshape(E)


def kernel(x_common, src, dst, base_w, enc_W0, enc_b0, enc_W1, enc_b1,
           u0_W0, u0_b0, u0_W1, u0_b1, u1_W0, u1_b0, u1_W1, u1_b1,
           ln0_g, ln0_b, ln1_g, ln1_b, g_W0, g_b0, g_W1, g_b1,
           rho_raw, toU_W, toU_b):
    f32 = jnp.float32
    gate = jax.nn.sigmoid(g_b1[0])
    rho0 = jax.nn.sigmoid(rho_raw[0])
    cst = (gate * rho0 * rho0).astype(f32).reshape(1, 1)

    pad = EP - E
    pad_idx = (jnp.arange(pad, dtype=jnp.int32) * 37) % N
    srcp = jnp.concatenate([src.astype(jnp.int32), pad_idx])
    dstp = jnp.concatenate([dst.astype(jnp.int32), pad_idx])
    wp = jnp.concatenate([base_w.astype(f32), jnp.zeros((pad,), f32)])
    zeros_acc = jnp.zeros((NP, HD), f32)

    xp = jnp.zeros((NP, C), f32).at[:N].set(x_common.astype(f32))
    eb0 = enc_b0.reshape(1, H)
    eb1 = enc_b1.reshape(1, H)

    h0 = _encoder(xp, enc_W0, eb0, enc_W1, eb1)

    msum1 = _sc_msgpass(h0, srcp, dstp, wp, zeros_acc)
    h1 = _update(msum1, h0, cst, u0_W0, u0_b0.reshape(1, H),
                 u0_W1, u0_b1.reshape(1, H), ln0_g.reshape(1, H),
                 ln0_b.reshape(1, H), toU_W, toU_b.reshape(1, K),
                 include_u=False)[0]

    msum2 = _sc_msgpass(h1, srcp, dstp, wp, zeros_acc)
    h2, U = _update(msum2, h1, cst, u1_W0, u1_b0.reshape(1, H),
                    u1_W1, u1_b1.reshape(1, H), ln1_g.reshape(1, H),
                    ln1_b.reshape(1, H), toU_W, toU_b.reshape(1, K),
                    include_u=True)

    w_out = _wout(base_w.astype(f32), cst)
    return U[:N], h2[:N, :H], w_out
